# Initial kernel scaffold; baseline (speedup 1.0000x reference)
import jax
import jax.numpy as jnp
from jax.experimental import pallas as pl


def kernel(x):
    raise NotImplementedError("write your pallas kernel here")


# speedup vs baseline: 9.0748x; 9.0748x over previous
import jax
import jax.numpy as jnp
from jax.experimental import pallas as pl

KNN_K = 20
BR = 256
BIG = 3.0e38


def _knn_body(xr_ref, xa_ref, out_ref):
    xr = xr_ref[0]
    xa = xa_ref[0]
    n = xa.shape[0]
    inner = jax.lax.dot_general(
        xr, xa, (((1,), (1,)), ((), ())),
        preferred_element_type=jnp.float32)
    xxr = jnp.sum(xr * xr, axis=1, keepdims=True)
    xxa = jnp.sum(xa * xa, axis=1)
    d = (xxr - 2.0 * inner) + xxa[None, :]

    iota = jax.lax.broadcasted_iota(jnp.int32, d.shape, 1).astype(jnp.float32)
    cols = []
    for _ in range(KNN_K):
        m = jnp.min(d, axis=1, keepdims=True)
        cand = jnp.where(d == m, iota, jnp.float32(n))
        j = jnp.min(cand, axis=1, keepdims=True)
        cols.append(j)
        d = jnp.where(cand == j, BIG, d)
    out_ref[0] = jnp.concatenate(cols, axis=1).astype(jnp.int32)


def kernel(x):
    b, n, dd = x.shape
    idx = pl.pallas_call(
        _knn_body,
        grid=(b, n // BR),
        in_specs=[
            pl.BlockSpec((1, BR, dd), lambda bi, ri: (bi, ri, 0)),
            pl.BlockSpec((1, n, dd), lambda bi, ri: (bi, 0, 0)),
        ],
        out_specs=pl.BlockSpec((1, BR, KNN_K), lambda bi, ri: (bi, ri, 0)),
        out_shape=jax.ShapeDtypeStruct((b, n, KNN_K), jnp.int32),
    )(x, x)
    return (x, idx)
